# baseline (speedup 1.0000x reference)
import functools

import jax
import jax.numpy as jnp
from jax import lax
from jax.experimental import pallas as pl
from jax.experimental.pallas import tpu as pltpu
from jax.experimental.pallas import tpu_sc as plsc

S = 16384
HL = 1024
HR = 256
W = HL + 2 * HR
L = 16

NC = 2
NS = 16
NW = NC * NS
BW = S // NW
RPC = 16
NCH = BW // RPC


@functools.partial(
    pl.kernel,
    mesh=plsc.VectorSubcoreMesh(core_axis_name="c", subcore_axis_name="s"),
    compiler_params=pltpu.CompilerParams(needs_layout_passes=False),
    out_type=jax.ShapeDtypeStruct((S, W), jnp.float32),
    scratch_types=[
        pltpu.SMEM((BW,), jnp.int32),
        pltpu.SMEM((BW,), jnp.int32),
        pltpu.SMEM((BW,), jnp.int32),
        pltpu.VMEM((3 * BW,), jnp.int32),
        pltpu.VMEM((5 * HL,), jnp.float32),
        pltpu.VMEM((50 * HR,), jnp.float32),
        pltpu.VMEM((50 * HR,), jnp.float32),
        pltpu.VMEM((RPC, W), jnp.float32),
        pltpu.VMEM((RPC, W), jnp.float32),
        pltpu.SemaphoreType.DMA,
        pltpu.SemaphoreType.DMA,
        pltpu.SemaphoreType.DMA,
    ],
)
def _sc_embed(lab_i_hbm, row_i_hbm, col_i_hbm, lab_w_hbm, row_w_hbm,
              col_w_hbm, out_hbm, lab_i, row_i, col_i, idx_v, lab_w_v,
              row_w_v, col_w_v, buf0, buf1, ssem0, ssem1, stsem):
    wid = lax.axis_index("s") * NC + lax.axis_index("c")
    iota = lax.iota(jnp.int32, L)

    i1 = pltpu.async_copy(lab_i_hbm.at[wid], idx_v.at[pl.ds(0, BW)], stsem)
    i2 = pltpu.async_copy(row_i_hbm.at[wid], idx_v.at[pl.ds(BW, BW)], stsem)
    i3 = pltpu.async_copy(col_i_hbm.at[wid], idx_v.at[pl.ds(2 * BW, BW)],
                          stsem)
    t1 = pltpu.async_copy(lab_w_hbm, lab_w_v, stsem)
    t2 = pltpu.async_copy(row_w_hbm, row_w_v, stsem)
    t3 = pltpu.async_copy(col_w_hbm, col_w_v, stsem)
    i1.wait()
    i2.wait()
    i3.wait()

    def to_smem(seg, sm):
        @plsc.parallel_loop(0, BW // L, unroll=2)
        def vbody(vc):
            v = idx_v[pl.ds(seg * BW + vc * L, L)]
            for r in range(L):
                sm[vc * L + r] = jnp.sum(jnp.where(iota == r, v, 0))

    to_smem(0, lab_i)
    to_smem(1, row_i)
    to_smem(2, col_i)
    t1.wait()
    t2.wait()
    t3.wait()

    PD = 16

    def assemble(c, buf):
        @plsc.parallel_loop(0, RPC, unroll=2)
        def rowbody(r):
            i = c * RPC + r
            ls = lab_i[i] * HL
            rs = row_i[i] * HR
            cs = col_i[i] * HR
            groups = (
                [(lab_w_v, ls + L * k, L * k) for k in range(HL // L)]
                + [(row_w_v, rs + L * k, HL + L * k)
                   for k in range(HR // L)]
                + [(col_w_v, cs + L * k, HL + HR + L * k)
                   for k in range(HR // L)])
            n = len(groups)
            loaded = [ref[pl.ds(base, L)] for ref, base, _ in groups[:PD]]
            for gi in range(PD, n):
                _, _, off = groups[gi - PD]
                buf[r, pl.ds(off, L)] = loaded[gi - PD]
                ref, base, _ = groups[gi]
                loaded.append(ref[pl.ds(base, L)])
            for gi in range(n - PD, n):
                _, _, off = groups[gi]
                buf[r, pl.ds(off, L)] = loaded[gi]

    def out_slab(c):
        return out_hbm.at[pl.ds(wid * BW + c * RPC, RPC)]

    def body(g, carry):
        c0 = 2 * g
        c1 = c0 + 1

        @pl.when(g > 0)
        def _():
            pltpu.make_async_copy(buf0, out_slab(c0 - 2), ssem0).wait()

        assemble(c0, buf0)
        pltpu.async_copy(buf0, out_slab(c0), ssem0)

        @pl.when(g > 0)
        def _():
            pltpu.make_async_copy(buf1, out_slab(c1 - 2), ssem1).wait()

        assemble(c1, buf1)
        pltpu.async_copy(buf1, out_slab(c1), ssem1)
        return carry

    lax.fori_loop(0, NCH // 2, body, 0)
    pltpu.make_async_copy(buf0, out_slab(NCH - 2), ssem0).wait()
    pltpu.make_async_copy(buf1, out_slab(NCH - 1), ssem1).wait()


def kernel(label, label_logits, row_id, column_id, epoch, label_emb_w,
           row_emb_w, col_emb_w):
    del label_logits, epoch
    lab_i = label.astype(jnp.int32).reshape(NW, BW)
    row_i = row_id.astype(jnp.int32).reshape(NW, BW)
    col_i = column_id.astype(jnp.int32).reshape(NW, BW)
    return _sc_embed(lab_i, row_i, col_i, label_emb_w.reshape(-1),
                     row_emb_w.reshape(-1), col_emb_w.reshape(-1))


# speedup vs baseline: 1.0553x; 1.0553x over previous
import functools

import jax
import jax.numpy as jnp
from jax import lax
from jax.experimental import pallas as pl
from jax.experimental.pallas import tpu as pltpu
from jax.experimental.pallas import tpu_sc as plsc

S = 16384
HL = 1024
HR = 256
W = HL + 2 * HR
L = 16

NC = 2
NS = 16
NW = NC * NS
BW = S // NW
RPC = 16
NCH = BW // RPC


@functools.partial(
    pl.kernel,
    mesh=plsc.VectorSubcoreMesh(core_axis_name="c", subcore_axis_name="s"),
    compiler_params=pltpu.CompilerParams(needs_layout_passes=False),
    out_type=jax.ShapeDtypeStruct((S, W), jnp.float32),
    scratch_types=[
        pltpu.SMEM((BW,), jnp.int32),
        pltpu.SMEM((BW,), jnp.int32),
        pltpu.SMEM((BW,), jnp.int32),
        pltpu.VMEM((3 * BW,), jnp.int32),
        pltpu.VMEM((5 * HL,), jnp.float32),
        pltpu.VMEM((50 * HR,), jnp.float32),
        pltpu.VMEM((50 * HR,), jnp.float32),
        pltpu.VMEM((RPC, W), jnp.float32),
        pltpu.VMEM((RPC, W), jnp.float32),
        pltpu.SemaphoreType.DMA,
        pltpu.SemaphoreType.DMA,
        pltpu.SemaphoreType.DMA,
    ],
)
def _sc_embed(lab_i_hbm, row_i_hbm, col_i_hbm, lab_w_hbm, row_w_hbm,
              col_w_hbm, out_hbm, lab_i, row_i, col_i, idx_v, lab_w_v,
              row_w_v, col_w_v, buf0, buf1, ssem0, ssem1, stsem):
    wid = lax.axis_index("s") * NC + lax.axis_index("c")
    iota = lax.iota(jnp.int32, L)

    i1 = pltpu.async_copy(lab_i_hbm.at[wid], idx_v.at[pl.ds(0, BW)], stsem)
    i2 = pltpu.async_copy(row_i_hbm.at[wid], idx_v.at[pl.ds(BW, BW)], stsem)
    i3 = pltpu.async_copy(col_i_hbm.at[wid], idx_v.at[pl.ds(2 * BW, BW)],
                          stsem)
    t1 = pltpu.async_copy(lab_w_hbm, lab_w_v, stsem)
    t2 = pltpu.async_copy(row_w_hbm, row_w_v, stsem)
    t3 = pltpu.async_copy(col_w_hbm, col_w_v, stsem)
    i1.wait()
    i2.wait()
    i3.wait()

    def to_smem(seg, sm):
        @plsc.parallel_loop(0, BW // L, unroll=2)
        def vbody(vc):
            v = idx_v[pl.ds(seg * BW + vc * L, L)]
            for r in range(L):
                sm[vc * L + r] = jnp.sum(jnp.where(iota == r, v, 0))

    to_smem(0, lab_i)
    to_smem(1, row_i)
    to_smem(2, col_i)
    t1.wait()
    t2.wait()
    t3.wait()

    PD = 6

    def assemble(c, buf):
        @plsc.parallel_loop(0, RPC, unroll=2)
        def rowbody(r):
            i = c * RPC + r
            ls = lab_i[i] * HL
            rs = row_i[i] * HR
            cs = col_i[i] * HR
            groups = (
                [(lab_w_v, ls + L * k, L * k) for k in range(HL // L)]
                + [(row_w_v, rs + L * k, HL + L * k)
                   for k in range(HR // L)]
                + [(col_w_v, cs + L * k, HL + HR + L * k)
                   for k in range(HR // L)])
            n = len(groups)
            loaded = [ref[pl.ds(base, L)] for ref, base, _ in groups[:PD]]
            for gi in range(PD, n):
                _, _, off = groups[gi - PD]
                buf[r, pl.ds(off, L)] = loaded[gi - PD]
                ref, base, _ = groups[gi]
                loaded.append(ref[pl.ds(base, L)])
            for gi in range(n - PD, n):
                _, _, off = groups[gi]
                buf[r, pl.ds(off, L)] = loaded[gi]

    def out_slab(c):
        return out_hbm.at[pl.ds(wid * BW + c * RPC, RPC)]

    def body(g, carry):
        c0 = 2 * g
        c1 = c0 + 1

        @pl.when(g > 0)
        def _():
            pltpu.make_async_copy(buf0, out_slab(c0 - 2), ssem0).wait()

        assemble(c0, buf0)
        pltpu.async_copy(buf0, out_slab(c0), ssem0)

        @pl.when(g > 0)
        def _():
            pltpu.make_async_copy(buf1, out_slab(c1 - 2), ssem1).wait()

        assemble(c1, buf1)
        pltpu.async_copy(buf1, out_slab(c1), ssem1)
        return carry

    lax.fori_loop(0, NCH // 2, body, 0)
    pltpu.make_async_copy(buf0, out_slab(NCH - 2), ssem0).wait()
    pltpu.make_async_copy(buf1, out_slab(NCH - 1), ssem1).wait()


def kernel(label, label_logits, row_id, column_id, epoch, label_emb_w,
           row_emb_w, col_emb_w):
    del label_logits, epoch
    lab_i = label.astype(jnp.int32).reshape(NW, BW)
    row_i = row_id.astype(jnp.int32).reshape(NW, BW)
    col_i = column_id.astype(jnp.int32).reshape(NW, BW)
    return _sc_embed(lab_i, row_i, col_i, label_emb_w.reshape(-1),
                     row_emb_w.reshape(-1), col_emb_w.reshape(-1))


# speedup vs baseline: 1.0597x; 1.0042x over previous
import functools

import jax
import jax.numpy as jnp
from jax import lax
from jax.experimental import pallas as pl
from jax.experimental.pallas import tpu as pltpu
from jax.experimental.pallas import tpu_sc as plsc

S = 16384
HL = 1024
HR = 256
W = HL + 2 * HR
L = 16

NC = 2
NS = 16
NW = NC * NS
BW = S // NW
RPC = 16
NCH = BW // RPC


@functools.partial(
    pl.kernel,
    mesh=plsc.VectorSubcoreMesh(core_axis_name="c", subcore_axis_name="s"),
    compiler_params=pltpu.CompilerParams(needs_layout_passes=False),
    out_type=jax.ShapeDtypeStruct((S, W), jnp.float32),
    scratch_types=[
        pltpu.SMEM((BW,), jnp.int32),
        pltpu.SMEM((BW,), jnp.int32),
        pltpu.SMEM((BW,), jnp.int32),
        pltpu.VMEM((3 * BW,), jnp.int32),
        pltpu.VMEM((5 * HL,), jnp.float32),
        pltpu.VMEM((50 * HR,), jnp.float32),
        pltpu.VMEM((50 * HR,), jnp.float32),
        pltpu.VMEM((RPC, W), jnp.float32),
        pltpu.VMEM((RPC, W), jnp.float32),
        pltpu.SemaphoreType.DMA,
        pltpu.SemaphoreType.DMA,
        pltpu.SemaphoreType.DMA,
    ],
)
def _sc_embed(lab_i_hbm, row_i_hbm, col_i_hbm, lab_w_hbm, row_w_hbm,
              col_w_hbm, out_hbm, lab_i, row_i, col_i, idx_v, lab_w_v,
              row_w_v, col_w_v, buf0, buf1, ssem0, ssem1, stsem):
    wid = lax.axis_index("s") * NC + lax.axis_index("c")
    iota = lax.iota(jnp.int32, L)

    i1 = pltpu.async_copy(lab_i_hbm.at[wid], idx_v.at[pl.ds(0, BW)], stsem)
    i2 = pltpu.async_copy(row_i_hbm.at[wid], idx_v.at[pl.ds(BW, BW)], stsem)
    i3 = pltpu.async_copy(col_i_hbm.at[wid], idx_v.at[pl.ds(2 * BW, BW)],
                          stsem)
    t1 = pltpu.async_copy(lab_w_hbm, lab_w_v, stsem)
    t2 = pltpu.async_copy(row_w_hbm, row_w_v, stsem)
    t3 = pltpu.async_copy(col_w_hbm, col_w_v, stsem)
    i1.wait()
    i2.wait()
    i3.wait()

    def to_smem(seg, sm):
        @plsc.parallel_loop(0, BW // L, unroll=2)
        def vbody(vc):
            v = idx_v[pl.ds(seg * BW + vc * L, L)]
            for r in range(L):
                sm[vc * L + r] = jnp.sum(jnp.where(iota == r, v, 0))

    to_smem(0, lab_i)
    to_smem(1, row_i)
    to_smem(2, col_i)
    t1.wait()
    t2.wait()
    t3.wait()

    PD = 8

    def assemble(c, buf):
        @plsc.parallel_loop(0, RPC, unroll=1)
        def rowbody(r):
            i = c * RPC + r
            ls = lab_i[i] * HL
            rs = row_i[i] * HR
            cs = col_i[i] * HR
            groups = (
                [(lab_w_v, ls + L * k, L * k) for k in range(HL // L)]
                + [(row_w_v, rs + L * k, HL + L * k)
                   for k in range(HR // L)]
                + [(col_w_v, cs + L * k, HL + HR + L * k)
                   for k in range(HR // L)])
            n = len(groups)
            loaded = [ref[pl.ds(base, L)] for ref, base, _ in groups[:PD]]
            for gi in range(PD, n):
                _, _, off = groups[gi - PD]
                buf[r, pl.ds(off, L)] = loaded[gi - PD]
                ref, base, _ = groups[gi]
                loaded.append(ref[pl.ds(base, L)])
            for gi in range(n - PD, n):
                _, _, off = groups[gi]
                buf[r, pl.ds(off, L)] = loaded[gi]

    def out_slab(c):
        return out_hbm.at[pl.ds(wid * BW + c * RPC, RPC)]

    def body(g, carry):
        c0 = 2 * g
        c1 = c0 + 1

        @pl.when(g > 0)
        def _():
            pltpu.make_async_copy(buf0, out_slab(c0 - 2), ssem0).wait()

        assemble(c0, buf0)
        pltpu.async_copy(buf0, out_slab(c0), ssem0)

        @pl.when(g > 0)
        def _():
            pltpu.make_async_copy(buf1, out_slab(c1 - 2), ssem1).wait()

        assemble(c1, buf1)
        pltpu.async_copy(buf1, out_slab(c1), ssem1)
        return carry

    lax.fori_loop(0, NCH // 2, body, 0)
    pltpu.make_async_copy(buf0, out_slab(NCH - 2), ssem0).wait()
    pltpu.make_async_copy(buf1, out_slab(NCH - 1), ssem1).wait()


def kernel(label, label_logits, row_id, column_id, epoch, label_emb_w,
           row_emb_w, col_emb_w):
    del label_logits, epoch
    lab_i = label.astype(jnp.int32).reshape(NW, BW)
    row_i = row_id.astype(jnp.int32).reshape(NW, BW)
    col_i = column_id.astype(jnp.int32).reshape(NW, BW)
    return _sc_embed(lab_i, row_i, col_i, label_emb_w.reshape(-1),
                     row_emb_w.reshape(-1), col_emb_w.reshape(-1))


# speedup vs baseline: 1.0605x; 1.0008x over previous
import functools

import jax
import jax.numpy as jnp
from jax import lax
from jax.experimental import pallas as pl
from jax.experimental.pallas import tpu as pltpu
from jax.experimental.pallas import tpu_sc as plsc

S = 16384
HL = 1024
HR = 256
W = HL + 2 * HR
L = 16

NC = 2
NS = 16
NW = NC * NS
BW = S // NW
RPC = 16
NCH = BW // RPC


@functools.partial(
    pl.kernel,
    mesh=plsc.VectorSubcoreMesh(core_axis_name="c", subcore_axis_name="s"),
    compiler_params=pltpu.CompilerParams(needs_layout_passes=False),
    out_type=jax.ShapeDtypeStruct((S, W), jnp.float32),
    scratch_types=[
        pltpu.SMEM((BW,), jnp.int32),
        pltpu.SMEM((BW,), jnp.int32),
        pltpu.SMEM((BW,), jnp.int32),
        pltpu.VMEM((3 * BW,), jnp.int32),
        pltpu.VMEM((5 * HL,), jnp.float32),
        pltpu.VMEM((50 * HR,), jnp.float32),
        pltpu.VMEM((50 * HR,), jnp.float32),
        pltpu.VMEM((RPC, W), jnp.float32),
        pltpu.VMEM((RPC, W), jnp.float32),
        pltpu.SemaphoreType.DMA,
        pltpu.SemaphoreType.DMA,
        pltpu.SemaphoreType.DMA,
    ],
)
def _sc_embed(lab_i_hbm, row_i_hbm, col_i_hbm, lab_w_hbm, row_w_hbm,
              col_w_hbm, out_hbm, lab_i, row_i, col_i, idx_v, lab_w_v,
              row_w_v, col_w_v, buf0, buf1, ssem0, ssem1, stsem):
    wid = lax.axis_index("s") * NC + lax.axis_index("c")
    iota = lax.iota(jnp.int32, L)

    i1 = pltpu.async_copy(lab_i_hbm.at[wid], idx_v.at[pl.ds(0, BW)], stsem)
    i2 = pltpu.async_copy(row_i_hbm.at[wid], idx_v.at[pl.ds(BW, BW)], stsem)
    i3 = pltpu.async_copy(col_i_hbm.at[wid], idx_v.at[pl.ds(2 * BW, BW)],
                          stsem)
    t1 = pltpu.async_copy(lab_w_hbm, lab_w_v, stsem)
    t2 = pltpu.async_copy(row_w_hbm, row_w_v, stsem)
    t3 = pltpu.async_copy(col_w_hbm, col_w_v, stsem)
    i1.wait()
    i2.wait()
    i3.wait()

    def to_smem(seg, sm):
        @plsc.parallel_loop(0, BW // L, unroll=2)
        def vbody(vc):
            v = idx_v[pl.ds(seg * BW + vc * L, L)]
            for r in range(L):
                sm[vc * L + r] = jnp.sum(jnp.where(iota == r, v, 0))

    to_smem(0, lab_i)
    to_smem(1, row_i)
    to_smem(2, col_i)
    t1.wait()
    t2.wait()
    t3.wait()

    PD = 8

    def assemble(c, buf):
        @plsc.parallel_loop(0, RPC, unroll=1)
        def rowbody(r):
            i = c * RPC + r
            ls = lab_i[i] * HL
            rs = row_i[i] * HR
            cs = col_i[i] * HR
            groups = (
                [(lab_w_v, ls + L * k, L * k) for k in range(HL // L)]
                + [(row_w_v, rs + L * k, HL + L * k)
                   for k in range(HR // L)]
                + [(col_w_v, cs + L * k, HL + HR + L * k)
                   for k in range(HR // L)])
            n = len(groups)
            loaded = [ref[pl.ds(base, L)] for ref, base, _ in groups[:PD]]
            for gi in range(PD, n):
                _, _, off = groups[gi - PD]
                buf[r, pl.ds(off, L)] = loaded[gi - PD]
                ref, base, _ = groups[gi]
                loaded.append(ref[pl.ds(base, L)])
            for gi in range(n - PD, n):
                _, _, off = groups[gi]
                buf[r, pl.ds(off, L)] = loaded[gi]

    H = RPC // 2

    def wr(buf, c, sem):
        pltpu.async_copy(
            buf.at[pl.ds(0, H)],
            out_hbm.at[pl.ds(wid * BW + c * RPC, H)], sem)
        pltpu.async_copy(
            buf.at[pl.ds(H, H)],
            out_hbm.at[pl.ds(wid * BW + c * RPC + H, H)], sem)

    def wr_wait(buf, c, sem):
        pltpu.make_async_copy(
            buf.at[pl.ds(0, H)],
            out_hbm.at[pl.ds(wid * BW + c * RPC, H)], sem).wait()
        pltpu.make_async_copy(
            buf.at[pl.ds(H, H)],
            out_hbm.at[pl.ds(wid * BW + c * RPC + H, H)], sem).wait()

    def body(g, carry):
        c0 = 2 * g
        c1 = c0 + 1

        @pl.when(g > 0)
        def _():
            wr_wait(buf0, c0 - 2, ssem0)

        assemble(c0, buf0)
        wr(buf0, c0, ssem0)

        @pl.when(g > 0)
        def _():
            wr_wait(buf1, c1 - 2, ssem1)

        assemble(c1, buf1)
        wr(buf1, c1, ssem1)
        return carry

    lax.fori_loop(0, NCH // 2, body, 0)
    wr_wait(buf0, NCH - 2, ssem0)
    wr_wait(buf1, NCH - 1, ssem1)


def kernel(label, label_logits, row_id, column_id, epoch, label_emb_w,
           row_emb_w, col_emb_w):
    del label_logits, epoch
    lab_i = label.astype(jnp.int32).reshape(NW, BW)
    row_i = row_id.astype(jnp.int32).reshape(NW, BW)
    col_i = column_id.astype(jnp.int32).reshape(NW, BW)
    return _sc_embed(lab_i, row_i, col_i, label_emb_w.reshape(-1),
                     row_emb_w.reshape(-1), col_emb_w.reshape(-1))


# speedup vs baseline: 1.1139x; 1.0503x over previous
import functools

import jax
import jax.numpy as jnp
from jax import lax
from jax.experimental import pallas as pl
from jax.experimental.pallas import tpu as pltpu
from jax.experimental.pallas import tpu_sc as plsc

S = 16384
HL = 1024
HR = 256
W = HL + 2 * HR
L = 16

NC = 2
NS = 16
NW = NC * NS
BW = S // NW
RPC = 16
NCH = BW // RPC


@functools.partial(
    pl.kernel,
    mesh=plsc.VectorSubcoreMesh(core_axis_name="c", subcore_axis_name="s"),
    compiler_params=pltpu.CompilerParams(needs_layout_passes=False),
    out_type=jax.ShapeDtypeStruct((S, W), jnp.float32),
    scratch_types=[
        pltpu.SMEM((BW,), jnp.int32),
        pltpu.SMEM((BW,), jnp.int32),
        pltpu.SMEM((BW,), jnp.int32),
        pltpu.VMEM((BW,), jnp.int32),
        pltpu.VMEM((5 * HL,), jnp.float32),
        pltpu.VMEM((50 * HR,), jnp.float32),
        pltpu.VMEM((50 * HR,), jnp.float32),
        pltpu.VMEM((RPC, W), jnp.float32),
        pltpu.VMEM((RPC, W), jnp.float32),
        pltpu.SemaphoreType.DMA,
        pltpu.SemaphoreType.DMA,
        pltpu.SemaphoreType.DMA,
    ],
)
def _sc_embed(pk_i_hbm, lab_w_hbm, row_w_hbm,
              col_w_hbm, out_hbm, lab_i, row_i, col_i, idx_v, lab_w_v,
              row_w_v, col_w_v, buf0, buf1, ssem0, ssem1, stsem):
    wid = lax.axis_index("s") * NC + lax.axis_index("c")
    iota = lax.iota(jnp.int32, L)

    i1 = pltpu.async_copy(pk_i_hbm.at[wid], idx_v, stsem)
    t1 = pltpu.async_copy(lab_w_hbm, lab_w_v, stsem)
    t2 = pltpu.async_copy(row_w_hbm, row_w_v, stsem)
    t3 = pltpu.async_copy(col_w_hbm, col_w_v, stsem)
    i1.wait()

    @plsc.parallel_loop(0, BW // L, unroll=2)
    def vbody(vc):
        v = idx_v[pl.ds(vc * L, L)]
        for r in range(L):
            s = jnp.sum(jnp.where(iota == r, v, 0))
            lab_i[vc * L + r] = s >> 12
            row_i[vc * L + r] = (s >> 6) & 0x3F
            col_i[vc * L + r] = s & 0x3F

    t1.wait()
    t2.wait()
    t3.wait()

    PD = 8

    def assemble(c, buf):
        @plsc.parallel_loop(0, RPC, unroll=1)
        def rowbody(r):
            i = c * RPC + r
            ls = lab_i[i] * HL
            rs = row_i[i] * HR
            cs = col_i[i] * HR
            groups = (
                [(lab_w_v, ls + L * k, L * k) for k in range(HL // L)]
                + [(row_w_v, rs + L * k, HL + L * k)
                   for k in range(HR // L)]
                + [(col_w_v, cs + L * k, HL + HR + L * k)
                   for k in range(HR // L)])
            n = len(groups)
            loaded = [ref[pl.ds(base, L)] for ref, base, _ in groups[:PD]]
            for gi in range(PD, n):
                _, _, off = groups[gi - PD]
                buf[r, pl.ds(off, L)] = loaded[gi - PD]
                ref, base, _ = groups[gi]
                loaded.append(ref[pl.ds(base, L)])
            for gi in range(n - PD, n):
                _, _, off = groups[gi]
                buf[r, pl.ds(off, L)] = loaded[gi]

    H = RPC // 2

    def wr(buf, c, sem):
        pltpu.async_copy(
            buf.at[pl.ds(0, H)],
            out_hbm.at[pl.ds(wid * BW + c * RPC, H)], sem)
        pltpu.async_copy(
            buf.at[pl.ds(H, H)],
            out_hbm.at[pl.ds(wid * BW + c * RPC + H, H)], sem)

    def wr_wait(buf, c, sem):
        pltpu.make_async_copy(
            buf.at[pl.ds(0, H)],
            out_hbm.at[pl.ds(wid * BW + c * RPC, H)], sem).wait()
        pltpu.make_async_copy(
            buf.at[pl.ds(H, H)],
            out_hbm.at[pl.ds(wid * BW + c * RPC + H, H)], sem).wait()

    def body(g, carry):
        c0 = 2 * g
        c1 = c0 + 1

        @pl.when(g > 0)
        def _():
            wr_wait(buf0, c0 - 2, ssem0)

        assemble(c0, buf0)
        wr(buf0, c0, ssem0)

        @pl.when(g > 0)
        def _():
            wr_wait(buf1, c1 - 2, ssem1)

        assemble(c1, buf1)
        wr(buf1, c1, ssem1)
        return carry

    lax.fori_loop(0, NCH // 2, body, 0)
    wr_wait(buf0, NCH - 2, ssem0)
    wr_wait(buf1, NCH - 1, ssem1)


def kernel(label, label_logits, row_id, column_id, epoch, label_emb_w,
           row_emb_w, col_emb_w):
    del label_logits, epoch
    pk_i = ((label.astype(jnp.int32) << 12)
            | (row_id.astype(jnp.int32) << 6)
            | column_id.astype(jnp.int32)).reshape(NW, BW)
    return _sc_embed(pk_i, label_emb_w.reshape(-1),
                     row_emb_w.reshape(-1), col_emb_w.reshape(-1))
